# Initial kernel scaffold; baseline (speedup 1.0000x reference)
import functools

import jax
import jax.numpy as jnp
from jax import lax
from jax.experimental import pallas as pl
from jax.experimental.pallas import tpu as pltpu
from jax.experimental.pallas import tpu_sc as plsc

N_NODES = 10000
N_EDGES = 320000
NFEAT = 128
NHID = 128
NCLASS = 64

NC = 2
NS = 16
NW = NC * NS
W_EDGE = 80
N_WIN = N_EDGES // W_EDGE
WIN_PER_WORKER = N_WIN // NW
ROWS_PER_SUB = N_NODES // NS


def _sc_segment_sum(width):
    mesh = plsc.VectorSubcoreMesh(core_axis_name="c", subcore_axis_name="s")

    @functools.partial(
        pl.kernel,
        mesh=mesh,
        out_type=jax.ShapeDtypeStruct((NC, N_NODES, width), jnp.float32),
        scratch_types=[
            pltpu.VMEM_SHARED((N_NODES, width), jnp.float32),
            pltpu.VMEM((1, W_EDGE), jnp.int32),
            pltpu.VMEM((1, W_EDGE), jnp.int32),
            pltpu.VMEM((W_EDGE, width), jnp.float32),
            pltpu.SemaphoreType.DMA,
        ],
    )
    def k(table_hbm, src_hbm, dst_hbm, zeros_hbm, out_hbm,
          acc, idx_s, idx_d, rows, sem):
        c = lax.axis_index("c")
        s = lax.axis_index("s")
        wid = c * NS + s

        pltpu.sync_copy(zeros_hbm, acc.at[pl.ds(s * ROWS_PER_SUB, ROWS_PER_SUB)])
        plsc.subcore_barrier()

        @pl.loop(0, WIN_PER_WORKER)
        def _(j):
            row = wid * WIN_PER_WORKER + j
            pltpu.sync_copy(src_hbm.at[pl.ds(row, 1)], idx_s)
            pltpu.sync_copy(dst_hbm.at[pl.ds(row, 1)], idx_d)
            pltpu.async_copy(table_hbm.at[idx_s.at[0]], rows, sem).wait()
            pltpu.sync_copy(rows, acc.at[idx_d.at[0]], add=True)

        plsc.subcore_barrier()
        pltpu.sync_copy(
            acc.at[pl.ds(s * ROWS_PER_SUB, ROWS_PER_SUB)],
            out_hbm.at[c].at[pl.ds(s * ROWS_PER_SUB, ROWS_PER_SUB)],
        )

    return k


def _stage_a(x_ref, w1l_ref, w1r_ref, ya_ref, r1_ref):
    xw = jnp.dot(x_ref[...], w1l_ref[...], preferred_element_type=jnp.float32)
    ones_col = (lax.broadcasted_iota(jnp.int32, (x_ref.shape[0], 16), 1) == 0)
    ya_ref[...] = jnp.concatenate(
        [xw, ones_col.astype(jnp.float32)], axis=1)
    r1_ref[...] = jnp.dot(x_ref[...], w1r_ref[...],
                          preferred_element_type=jnp.float32)


def _stage_c(p0_ref, p1_ref, r1_ref, b1_ref, w2l_ref, w2r_ref, b2_ref,
             y2_ref, r2_ref, deg_ref):
    acc = p0_ref[...] + p1_ref[...]
    deg = jnp.clip(acc[:, NFEAT:NFEAT + 1], 1.0, None)
    h = jnp.maximum(acc[:, :NFEAT] / deg + b1_ref[...] + r1_ref[...], 0.0)
    y2_ref[...] = jnp.dot(h, w2l_ref[...], preferred_element_type=jnp.float32)
    r2_ref[...] = jnp.dot(h, w2r_ref[...],
                          preferred_element_type=jnp.float32) + b2_ref[...]
    deg_ref[...] = deg


def _stage_e(q0_ref, q1_ref, deg_ref, r2_ref, out_ref):
    out_ref[...] = (q0_ref[...] + q1_ref[...]) / deg_ref[...] + r2_ref[...]


@jax.jit
def kernel(x, adj, W1l, b1, W1r, W2l, b2, W2r):
    src = adj[0].astype(jnp.int32).reshape(N_WIN, W_EDGE)
    dst = adj[1].astype(jnp.int32).reshape(N_WIN, W_EDGE)
    b1r = b1.reshape(1, NHID)
    b2r = b2.reshape(1, NCLASS)

    ya, r1 = pl.pallas_call(
        _stage_a,
        out_shape=[
            jax.ShapeDtypeStruct((N_NODES, NFEAT + 16), jnp.float32),
            jax.ShapeDtypeStruct((N_NODES, NHID), jnp.float32),
        ],
    )(x, W1l, W1r)

    z144 = jnp.zeros((ROWS_PER_SUB, NFEAT + 16), jnp.float32)
    p = _sc_segment_sum(NFEAT + 16)(ya, src, dst, z144)

    y2, r2, deg = pl.pallas_call(
        _stage_c,
        out_shape=[
            jax.ShapeDtypeStruct((N_NODES, NCLASS), jnp.float32),
            jax.ShapeDtypeStruct((N_NODES, NCLASS), jnp.float32),
            jax.ShapeDtypeStruct((N_NODES, 1), jnp.float32),
        ],
    )(p[0], p[1], r1, b1r, W2l, W2r, b2r)

    z64 = jnp.zeros((ROWS_PER_SUB, NCLASS), jnp.float32)
    q = _sc_segment_sum(NCLASS)(y2, src, dst, z64)

    out = pl.pallas_call(
        _stage_e,
        out_shape=jax.ShapeDtypeStruct((N_NODES, NCLASS), jnp.float32),
    )(q[0], q[1], deg, r2)

    return out


# speedup vs baseline: 5.2789x; 5.2789x over previous
import functools

import jax
import jax.numpy as jnp
from jax import lax
from jax.experimental import pallas as pl
from jax.experimental.pallas import tpu as pltpu
from jax.experimental.pallas import tpu_sc as plsc

N_NODES = 10000
N_EDGES = 320000
NFEAT = 128
NHID = 128
NCLASS = 64

NC = 2
NS = 16
NW = NC * NS
W_EDGE = 80
N_WIN = N_EDGES // W_EDGE
WIN_PER_WORKER = N_WIN // NW
N_PAD = 10240
ROWS_PER_SUB = N_PAD // NS

DEG_EB = 2000
DEG_NB = N_EDGES // DEG_EB


def _sc_segment_sum(width):
    mesh = plsc.VectorSubcoreMesh(core_axis_name="c", subcore_axis_name="s")

    @functools.partial(
        pl.kernel,
        mesh=mesh,
        out_type=jax.ShapeDtypeStruct((NC, N_PAD, width), jnp.float32),
        scratch_types=[
            pltpu.VMEM_SHARED((N_PAD, width), jnp.float32),
            pltpu.VMEM((1, W_EDGE), jnp.int32),
            pltpu.VMEM((1, W_EDGE), jnp.int32),
            pltpu.VMEM((W_EDGE, width), jnp.float32),
            pltpu.SemaphoreType.DMA,
        ],
    )
    def k(table_hbm, src_hbm, dst_hbm, zeros_hbm, out_hbm,
          acc, idx_s, idx_d, rows, sem):
        c = lax.axis_index("c")
        s = lax.axis_index("s")
        wid = c * NS + s

        pltpu.sync_copy(zeros_hbm, acc.at[pl.ds(s * ROWS_PER_SUB, ROWS_PER_SUB)])
        plsc.subcore_barrier()

        @pl.loop(0, WIN_PER_WORKER)
        def _(j):
            row = wid * WIN_PER_WORKER + j
            pltpu.sync_copy(src_hbm.at[pl.ds(row, 1)], idx_s)
            pltpu.sync_copy(dst_hbm.at[pl.ds(row, 1)], idx_d)
            pltpu.async_copy(table_hbm.at[idx_s.at[0]], rows, sem).wait()
            pltpu.sync_copy(rows, acc.at[idx_d.at[0]], add=True)

        plsc.subcore_barrier()
        pltpu.sync_copy(
            acc.at[pl.ds(s * ROWS_PER_SUB, ROWS_PER_SUB)],
            out_hbm.at[c].at[pl.ds(s * ROWS_PER_SUB, ROWS_PER_SUB)],
        )

    return k


def _stage_a(x_ref, w1l_ref, w1r_ref, y1_ref, r1_ref):
    y1_ref[...] = jnp.dot(x_ref[...], w1l_ref[...],
                          preferred_element_type=jnp.float32)
    r1_ref[...] = jnp.dot(x_ref[...], w1r_ref[...],
                          preferred_element_type=jnp.float32)


def _deg_kernel(dst_ref, out_ref):
    @pl.when(pl.program_id(0) == 0)
    def _():
        out_ref[...] = jnp.zeros_like(out_ref)

    d = dst_ref[0, 0, :]
    cols = lax.broadcasted_iota(jnp.int32, (DEG_EB, 128), 1)
    hi = ((d[:, None] >> 7) == cols).astype(jnp.float32)
    lo = ((d[:, None] & 127) == cols).astype(jnp.float32)
    out_ref[...] += lax.dot_general(
        hi, lo, (((0,), (0,)), ((), ())),
        preferred_element_type=jnp.float32)


def _stage_c(p0_ref, p1_ref, r1_ref, b1_ref, w2r_ref, b2_ref, deg_ref,
             h_ref, r2_ref):
    acc = p0_ref[...] + p1_ref[...]
    deg = jnp.clip(deg_ref[...], 1.0, None)
    h = jnp.maximum(acc / deg + b1_ref[...] + r1_ref[...], 0.0)
    h_ref[...] = h
    r2_ref[...] = jnp.dot(h, w2r_ref[...],
                          preferred_element_type=jnp.float32) + b2_ref[...]


def _stage_e(q0_ref, q1_ref, deg_ref, w2l_ref, r2_ref, out_ref):
    deg = jnp.clip(deg_ref[...], 1.0, None)
    mean2 = (q0_ref[...] + q1_ref[...]) / deg
    out_ref[...] = jnp.dot(mean2, w2l_ref[...],
                           preferred_element_type=jnp.float32) + r2_ref[...]


@jax.jit
def kernel(x, adj, W1l, b1, W1r, W2l, b2, W2r):
    src = adj[0].astype(jnp.int32).reshape(N_WIN, W_EDGE)
    dst = adj[1].astype(jnp.int32).reshape(N_WIN, W_EDGE)
    dst3 = adj[1].astype(jnp.int32).reshape(DEG_NB, 1, DEG_EB)
    b1r = b1.reshape(1, NHID)
    b2r = b2.reshape(1, NCLASS)
    zeros = jnp.zeros((ROWS_PER_SUB, NFEAT), jnp.float32)

    y1, r1 = pl.pallas_call(
        _stage_a,
        out_shape=[
            jax.ShapeDtypeStruct((N_NODES, NHID), jnp.float32),
            jax.ShapeDtypeStruct((N_NODES, NHID), jnp.float32),
        ],
    )(x, W1l, W1r)

    p = _sc_segment_sum(NFEAT)(y1, src, dst, zeros)

    deg_mat = pl.pallas_call(
        _deg_kernel,
        grid=(DEG_NB,),
        in_specs=[pl.BlockSpec((1, 1, DEG_EB), lambda i: (i, 0, 0))],
        out_specs=pl.BlockSpec((128, 128), lambda i: (0, 0)),
        out_shape=jax.ShapeDtypeStruct((128, 128), jnp.float32),
    )(dst3)
    deg = deg_mat.reshape(-1, 1)[:N_NODES]

    h, r2 = pl.pallas_call(
        _stage_c,
        out_shape=[
            jax.ShapeDtypeStruct((N_NODES, NHID), jnp.float32),
            jax.ShapeDtypeStruct((N_NODES, NCLASS), jnp.float32),
        ],
    )(p[0, :N_NODES], p[1, :N_NODES], r1, b1r, W2r, b2r, deg)

    q = _sc_segment_sum(NHID)(h, src, dst, zeros)

    out = pl.pallas_call(
        _stage_e,
        out_shape=jax.ShapeDtypeStruct((N_NODES, NCLASS), jnp.float32),
    )(q[0, :N_NODES], q[1, :N_NODES], deg, W2l, r2)

    return out
